# baseline (speedup 1.0000x reference)
import functools

import jax
import jax.numpy as jnp
from jax import lax
from jax.experimental import pallas as pl
from jax.experimental.pallas import tpu as pltpu
from jax.experimental.pallas import tpu_sc as plsc

NC = 2
NS = 16
NW = NC * NS
K = 128

_mesh = plsc.VectorSubcoreMesh(core_axis_name="c", subcore_axis_name="s")


def _make_deg_kernel(CH, ACC_N):
    RPT = ACC_N // NS
    CB = 8

    @functools.partial(
        pl.kernel, mesh=_mesh,
        out_type=[jax.ShapeDtypeStruct((NC, ACC_N), jnp.float32),
                  jax.ShapeDtypeStruct((NC, ACC_N), jnp.float32)],
        scratch_types=[
            pltpu.VMEM((CH, K), jnp.int32),
            pltpu.VMEM((CH, K), jnp.int32),
            pltpu.VMEM((K,), jnp.float32),
            pltpu.VMEM((RPT,), jnp.float32),
            pltpu.VMEM_SHARED((ACC_N,), jnp.float32),
            pltpu.VMEM_SHARED((ACC_N,), jnp.float32),
            pltpu.SemaphoreType.DMA,
            pltpu.SemaphoreType.DMA,
        ])
    def deg_kernel(src3_hbm, dst3_hbm, do_part, di_part,
                   sslab, dslab, ones, zs, acc_o, acc_i, sem_o, sem_i):
        cid = lax.axis_index("c")
        sid = lax.axis_index("s")
        wid = cid * NS + sid
        pltpu.sync_copy(src3_hbm.at[wid], sslab)
        pltpu.sync_copy(dst3_hbm.at[wid], dslab)
        for j in range(K // 16):
            ones[pl.ds(16 * j, 16)] = jnp.ones((16,), jnp.float32)
        for j in range(RPT // 16):
            zs[pl.ds(16 * j, 16)] = jnp.zeros((16,), jnp.float32)
        pltpu.sync_copy(zs, acc_o.at[pl.ds(sid * RPT, RPT)])
        pltpu.sync_copy(zs, acc_i.at[pl.ds(sid * RPT, RPT)])
        plsc.subcore_barrier()

        def body(b, carry):
            for j in range(CB):
                c = b * CB + j
                pltpu.async_copy(ones, acc_o.at[sslab.at[c]], sem_o, add=True)
                pltpu.async_copy(ones, acc_i.at[dslab.at[c]], sem_i, add=True)
            for j in range(CB):
                pltpu.make_async_copy(ones, acc_o.at[sslab.at[0]], sem_o).wait()
                pltpu.make_async_copy(ones, acc_i.at[dslab.at[0]], sem_i).wait()
            return carry

        lax.fori_loop(0, CH // CB, body, 0)
        plsc.subcore_barrier()
        pltpu.sync_copy(acc_o.at[pl.ds(sid * RPT, RPT)],
                        do_part.at[cid, pl.ds(sid * RPT, RPT)])
        pltpu.sync_copy(acc_i.at[pl.ds(sid * RPT, RPT)],
                        di_part.at[cid, pl.ds(sid * RPT, RPT)])

    return deg_kernel


def _make_prop128_kernel(CH0, ACC_N, D):
    RPT = ACC_N // NS
    H = K // 2
    PH = 2
    CHP = CH0 // PH

    @functools.partial(
        pl.kernel, mesh=_mesh,
        out_type=[jax.ShapeDtypeStruct((ACC_N, D), jnp.float32)],
        scratch_types=[
            pltpu.VMEM((CHP * K,), jnp.int32),
            pltpu.VMEM((K,), jnp.int32),
            pltpu.VMEM((K,), jnp.int32),
            pltpu.VMEM((K, D), jnp.float32),
            pltpu.VMEM((K, D), jnp.float32),
            pltpu.VMEM_SHARED((ACC_N, D), jnp.float32),
            pltpu.SemaphoreType.DMA,
            pltpu.SemaphoreType.DMA,
            pltpu.SemaphoreType.DMA,
            pltpu.SemaphoreType.DMA,
            pltpu.SemaphoreType.DMA,
        ])
    def prop_kernel(hs_hbm, srcf_hbm, dstf_hbm, part,
                    sslab, didx0, didx1, rows0, rows1, acc, g0, g1, d0, d1, zsem):
        cid = lax.axis_index("c")
        sid = lax.axis_index("s")

        @pl.when(cid == 0)
        def _():
            for r in range(16):
                for j in range(D // 16):
                    rows0[r, pl.ds(16 * j, 16)] = jnp.zeros((16,), jnp.float32)
            for k in range(RPT // 16):
                pltpu.async_copy(rows0.at[pl.ds(0, 16)],
                                 acc.at[pl.ds(sid * RPT + k * 16, 16)], zsem)
            for k in range(RPT // 16):
                pltpu.make_async_copy(rows0.at[pl.ds(0, 16)],
                                      acc.at[pl.ds(sid * RPT, 16)], zsem).wait()
            plsc.subcore_barrier()

            def fire(base_e, c, rows, gsem, didx, dsem):
                pltpu.async_copy(dstf_hbm.at[pl.ds(base_e + c * K, K)],
                                 didx, dsem)
                pltpu.async_copy(hs_hbm.at[sslab.at[pl.ds(c * K, H)]],
                                 rows.at[pl.ds(0, H)], gsem)
                pltpu.async_copy(hs_hbm.at[sslab.at[pl.ds(c * K + H, H)]],
                                 rows.at[pl.ds(H, H)], gsem)

            def wait_and_add(base_e, c, rows, gsem, didx, dsem):
                pltpu.make_async_copy(hs_hbm.at[sslab.at[pl.ds(c * K, H)]],
                                      rows.at[pl.ds(0, H)], gsem).wait()
                pltpu.make_async_copy(hs_hbm.at[sslab.at[pl.ds(c * K + H, H)]],
                                      rows.at[pl.ds(H, H)], gsem).wait()
                pltpu.make_async_copy(dstf_hbm.at[pl.ds(base_e + c * K, K)],
                                      didx, dsem).wait()
                pltpu.sync_copy(rows, acc.at[didx], add=True)

            for ph in range(PH):
                base_e = (sid * CH0 + ph * CHP) * K
                pltpu.sync_copy(srcf_hbm.at[pl.ds(base_e, CHP * K)], sslab)
                fire(base_e, 0, rows0, g0, didx0, d0)
                fire(base_e, 1, rows1, g1, didx1, d1)

                def body(i, carry):
                    c0 = 2 * i
                    c1 = c0 + 1
                    wait_and_add(base_e, c0, rows0, g0, didx0, d0)

                    @pl.when(c0 + 2 < CHP)
                    def _():
                        fire(base_e, c0 + 2, rows0, g0, didx0, d0)

                    wait_and_add(base_e, c1, rows1, g1, didx1, d1)

                    @pl.when(c1 + 2 < CHP)
                    def _():
                        fire(base_e, c1 + 2, rows1, g1, didx1, d1)

                    return carry

                lax.fori_loop(0, CHP // 2, body, 0)

            plsc.subcore_barrier()
            pltpu.sync_copy(acc.at[pl.ds(sid * RPT, RPT)],
                            part.at[pl.ds(sid * RPT, RPT)])

    return prop_kernel


def _make_prop1_kernel(CH, ACC_N, D):
    RPT = ACC_N // NS
    E_T = CH * K
    NB = 4

    @functools.partial(
        pl.kernel, mesh=_mesh,
        out_type=[jax.ShapeDtypeStruct((NC, ACC_N), jnp.float32)],
        scratch_types=[
            pltpu.VMEM((E_T,), jnp.int32),
            pltpu.VMEM((CH, K), jnp.int32),
            pltpu.VMEM((K,), jnp.float32),
            pltpu.VMEM((K,), jnp.float32),
            pltpu.VMEM((K,), jnp.float32),
            pltpu.VMEM((K,), jnp.float32),
            pltpu.VMEM((RPT,), jnp.float32),
            pltpu.VMEM_SHARED((ACC_N,), jnp.float32),
            pltpu.SemaphoreType.DMA,
            pltpu.SemaphoreType.DMA,
            pltpu.SemaphoreType.DMA,
            pltpu.SemaphoreType.DMA,
        ])
    def prop1_kernel(tflat_hbm, srcf_hbm, dst3_hbm, part,
                     sslab, dslab, v0, v1, v2, v3, zs, acc, s0, s1, s2, s3):
        cid = lax.axis_index("c")
        sid = lax.axis_index("s")
        wid = cid * NS + sid
        vals = [v0, v1, v2, v3]
        sems = [s0, s1, s2, s3]
        pltpu.sync_copy(srcf_hbm.at[pl.ds(wid * E_T, E_T)], sslab)
        pltpu.sync_copy(dst3_hbm.at[wid], dslab)

        def scale(c, carry):
            for j in range(K // 16):
                sl = pl.ds(c * K + 16 * j, 16)
                sslab[sl] = sslab[sl] * D
            return carry

        lax.fori_loop(0, CH, scale, 0)
        for j in range(RPT // 16):
            zs[pl.ds(16 * j, 16)] = jnp.zeros((16,), jnp.float32)
        pltpu.sync_copy(zs, acc.at[pl.ds(sid * RPT, RPT)])
        plsc.subcore_barrier()

        def fire(c, b):
            pltpu.async_copy(tflat_hbm.at[sslab.at[pl.ds(c * K, K)]],
                             vals[b], sems[b])

        def wait_and_add(c, b):
            pltpu.make_async_copy(tflat_hbm.at[sslab.at[pl.ds(c * K, K)]],
                                  vals[b], sems[b]).wait()
            pltpu.sync_copy(vals[b], acc.at[dslab.at[c]], add=True)

        for c in range(NB - 1):
            fire(c, c)

        def body(i, carry):
            for b in range(NB):
                c = NB * i + b
                wait_and_add(c, b)

                @pl.when(c + NB - 1 < CH)
                def _():
                    fire(c + NB - 1, (b + NB - 1) % NB)

            return carry

        lax.fori_loop(0, CH // NB, body, 0)
        plsc.subcore_barrier()
        pltpu.sync_copy(acc.at[pl.ds(sid * RPT, RPT)],
                        part.at[cid, pl.ds(sid * RPT, RPT)])

    return prop1_kernel


def _norm_from(deg):
    return jnp.where(deg > 0, lax.rsqrt(jnp.maximum(deg, 1.0)), 0.0)


def _mm1_body(x_ref, w_ref, dop_ref, out_ref):
    do = (dop_ref[0] + dop_ref[1]).reshape(-1)
    ns = _norm_from(do)
    h = jnp.dot(x_ref[...], w_ref[...], preferred_element_type=jnp.float32)
    out_ref[...] = h * ns[:, None]


def _mm2_body(aggp_ref, dip_ref, dop_ref, b1_ref, w2b_ref, out_ref):
    agg = aggp_ref[...]
    nd = _norm_from((dip_ref[0] + dip_ref[1]).reshape(-1))
    ns = _norm_from((dop_ref[0] + dop_ref[1]).reshape(-1))
    z = jnp.maximum(agg * nd[:, None] + b1_ref[0], 0.0)
    t = jnp.dot(z, w2b_ref[...], preferred_element_type=jnp.float32)
    out_ref[...] = t * ns[:, None]


def _fin_body(qp_ref, dip_ref, b2_ref, out_ref):
    q = qp_ref[0] + qp_ref[1]
    nd = _norm_from(dip_ref[0] + dip_ref[1])
    out_ref[...] = q * nd + b2_ref[0]


def kernel(x, edge_index, W1, b1, W2, b2):
    N, D = x.shape
    E = edge_index.shape[1]
    ACC_N = ((N + 1 + 2047) // 2048) * 2048
    CH = (E + NW * K - 1) // (NW * K)
    CH += (-CH) % 8
    EP = NW * K * CH
    NBR = 1024
    NB = ACC_N // NBR

    CHT = (E + NS * K - 1) // (NS * K)
    CH0 = CHT + ((-CHT) % 4)
    EPP = NS * K * CH0

    src = edge_index[0]
    dst = edge_index[1]
    pad = EP - E
    sent = jnp.full((pad,), N, jnp.int32)
    srcp = jnp.concatenate([src, sent])
    dstp = jnp.concatenate([dst, sent])
    src3 = srcp.reshape(NW, CH, K)
    dst3 = dstp.reshape(NW, CH, K)
    sent2 = jnp.full((EPP - E,), N, jnp.int32)
    srcf = jnp.concatenate([src, sent2])
    dstf = jnp.concatenate([dst, sent2])
    xp = jnp.pad(x, ((0, ACC_N - N), (0, 0)))

    dop, dip = _make_deg_kernel(CH, ACC_N)(src3, dst3)
    dop3 = dop.reshape(NC, ACC_N // 128, 128)
    dip3 = dip.reshape(NC, ACC_N // 128, 128)

    hs = pl.pallas_call(
        _mm1_body,
        grid=(NB,),
        in_specs=[
            pl.BlockSpec((NBR, D), lambda i: (i, 0)),
            pl.BlockSpec((D, D), lambda i: (0, 0)),
            pl.BlockSpec((NC, NBR // 128, 128), lambda i: (0, i, 0)),
        ],
        out_specs=pl.BlockSpec((NBR, D), lambda i: (i, 0)),
        out_shape=jax.ShapeDtypeStruct((ACC_N, D), jnp.float32),
    )(xp, W1, dop3)

    (part,) = _make_prop128_kernel(CH0, ACC_N, D)(hs, srcf, dstf)

    w2b = jnp.tile(W2, (1, D))
    t2d = pl.pallas_call(
        _mm2_body,
        grid=(NB,),
        in_specs=[
            pl.BlockSpec((NBR, D), lambda i: (i, 0)),
            pl.BlockSpec((NC, NBR // 128, 128), lambda i: (0, i, 0)),
            pl.BlockSpec((NC, NBR // 128, 128), lambda i: (0, i, 0)),
            pl.BlockSpec((1, D), lambda i: (0, 0)),
            pl.BlockSpec((D, D), lambda i: (0, 0)),
        ],
        out_specs=pl.BlockSpec((NBR, D), lambda i: (i, 0)),
        out_shape=jax.ShapeDtypeStruct((ACC_N, D), jnp.float32),
    )(part, dip3, dop3, b1.reshape(1, D), w2b)

    (part1,) = _make_prop1_kernel(CH, ACC_N, D)(t2d.reshape(-1), srcp, dst3)

    b2b = jnp.full((1, 128), b2[0], jnp.float32)
    outp = pl.pallas_call(
        _fin_body,
        grid=(ACC_N // 128 // 8,),
        in_specs=[
            pl.BlockSpec((NC, 8, 128), lambda i: (0, i, 0)),
            pl.BlockSpec((NC, 8, 128), lambda i: (0, i, 0)),
            pl.BlockSpec((1, 128), lambda i: (0, 0)),
        ],
        out_specs=pl.BlockSpec((8, 128), lambda i: (i, 0)),
        out_shape=jax.ShapeDtypeStruct((ACC_N // 128, 128), jnp.float32),
    )(part1.reshape(NC, ACC_N // 128, 128), dip3, b2b)

    return outp.reshape(-1)[:N].reshape(N, 1)


# speedup vs baseline: 1.5736x; 1.5736x over previous
import functools

import jax
import jax.numpy as jnp
from jax import lax
from jax.experimental import pallas as pl
from jax.experimental.pallas import tpu as pltpu
from jax.experimental.pallas import tpu_sc as plsc

NC = 2
NS = 16
NW = NC * NS
K = 128

_mesh = plsc.VectorSubcoreMesh(core_axis_name="c", subcore_axis_name="s")


def _make_deg_kernel(CH, ACC_N):
    RPT = ACC_N // NS
    CB = 8

    @functools.partial(
        pl.kernel, mesh=_mesh,
        out_type=[jax.ShapeDtypeStruct((NC, ACC_N), jnp.float32),
                  jax.ShapeDtypeStruct((NC, ACC_N), jnp.float32)],
        scratch_types=[
            pltpu.VMEM((CH, K), jnp.int32),
            pltpu.VMEM((CH, K), jnp.int32),
            pltpu.VMEM((K,), jnp.float32),
            pltpu.VMEM((RPT,), jnp.float32),
            pltpu.VMEM_SHARED((ACC_N,), jnp.float32),
            pltpu.VMEM_SHARED((ACC_N,), jnp.float32),
            pltpu.SemaphoreType.DMA,
            pltpu.SemaphoreType.DMA,
        ])
    def deg_kernel(src3_hbm, dst3_hbm, do_part, di_part,
                   sslab, dslab, ones, zs, acc_o, acc_i, sem_o, sem_i):
        cid = lax.axis_index("c")
        sid = lax.axis_index("s")
        wid = cid * NS + sid
        pltpu.sync_copy(src3_hbm.at[wid], sslab)
        pltpu.sync_copy(dst3_hbm.at[wid], dslab)
        for j in range(K // 16):
            ones[pl.ds(16 * j, 16)] = jnp.ones((16,), jnp.float32)
        for j in range(RPT // 16):
            zs[pl.ds(16 * j, 16)] = jnp.zeros((16,), jnp.float32)
        pltpu.sync_copy(zs, acc_o.at[pl.ds(sid * RPT, RPT)])
        pltpu.sync_copy(zs, acc_i.at[pl.ds(sid * RPT, RPT)])
        plsc.subcore_barrier()

        def body(b, carry):
            for j in range(CB):
                c = b * CB + j
                pltpu.async_copy(ones, acc_o.at[sslab.at[c]], sem_o, add=True)
                pltpu.async_copy(ones, acc_i.at[dslab.at[c]], sem_i, add=True)
            for j in range(CB):
                pltpu.make_async_copy(ones, acc_o.at[sslab.at[0]], sem_o).wait()
                pltpu.make_async_copy(ones, acc_i.at[dslab.at[0]], sem_i).wait()
            return carry

        lax.fori_loop(0, CH // CB, body, 0)
        plsc.subcore_barrier()
        pltpu.sync_copy(acc_o.at[pl.ds(sid * RPT, RPT)],
                        do_part.at[cid, pl.ds(sid * RPT, RPT)])
        pltpu.sync_copy(acc_i.at[pl.ds(sid * RPT, RPT)],
                        di_part.at[cid, pl.ds(sid * RPT, RPT)])

    return deg_kernel


def _make_prop128_kernel(CH0, CH1, ACC_N, D, KP):
    RPT = ACC_N // NS
    H = KP // 2
    CHM = max(CH0, CH1)

    @functools.partial(
        pl.kernel, mesh=_mesh,
        out_type=[jax.ShapeDtypeStruct((NC, ACC_N, D), jnp.float32)],
        scratch_types=[
            pltpu.VMEM((CHM * KP,), jnp.int32),
            pltpu.VMEM((KP,), jnp.int32),
            pltpu.VMEM((KP,), jnp.int32),
            pltpu.VMEM((KP, D), jnp.float32),
            pltpu.VMEM((KP, D), jnp.float32),
            pltpu.VMEM_SHARED((ACC_N, D), jnp.float32),
            pltpu.SemaphoreType.DMA,
            pltpu.SemaphoreType.DMA,
            pltpu.SemaphoreType.DMA,
            pltpu.SemaphoreType.DMA,
            pltpu.SemaphoreType.DMA,
        ])
    def prop_kernel(hs_hbm, srcf_hbm, dstf_hbm, part,
                    sslab, didx0, didx1, rows0, rows1, acc, g0, g1, d0, d1, zsem):
        cid = lax.axis_index("c")
        sid = lax.axis_index("s")
        for r in range(16):
            for j in range(D // 16):
                rows0[r, pl.ds(16 * j, 16)] = jnp.zeros((16,), jnp.float32)
        for k in range(RPT // 16):
            pltpu.async_copy(rows0.at[pl.ds(0, 16)],
                             acc.at[pl.ds(sid * RPT + k * 16, 16)], zsem)
        for k in range(RPT // 16):
            pltpu.make_async_copy(rows0.at[pl.ds(0, 16)],
                                  acc.at[pl.ds(sid * RPT, 16)], zsem).wait()
        plsc.subcore_barrier()

        def pipeline(CHW, base_c):
            base_e = base_c * KP
            pltpu.sync_copy(srcf_hbm.at[pl.ds(base_e, CHW * KP)],
                            sslab.at[pl.ds(0, CHW * KP)])

            def fire(c, rows, gsem, didx, dsem):
                pltpu.async_copy(dstf_hbm.at[pl.ds(base_e + c * KP, KP)],
                                 didx, dsem)
                pltpu.async_copy(hs_hbm.at[sslab.at[pl.ds(c * KP, H)]],
                                 rows.at[pl.ds(0, H)], gsem)
                pltpu.async_copy(hs_hbm.at[sslab.at[pl.ds(c * KP + H, H)]],
                                 rows.at[pl.ds(H, H)], gsem)

            def wait_and_add(c, rows, gsem, didx, dsem):
                pltpu.make_async_copy(hs_hbm.at[sslab.at[pl.ds(c * KP, H)]],
                                      rows.at[pl.ds(0, H)], gsem).wait()
                pltpu.make_async_copy(hs_hbm.at[sslab.at[pl.ds(c * KP + H, H)]],
                                      rows.at[pl.ds(H, H)], gsem).wait()
                pltpu.make_async_copy(dstf_hbm.at[pl.ds(base_e + c * KP, KP)],
                                      didx, dsem).wait()
                pltpu.sync_copy(rows, acc.at[didx], add=True)

            fire(0, rows0, g0, didx0, d0)
            fire(1, rows1, g1, didx1, d1)

            def body(i, carry):
                c0 = 2 * i
                c1 = c0 + 1
                wait_and_add(c0, rows0, g0, didx0, d0)

                @pl.when(c0 + 2 < CHW)
                def _():
                    fire(c0 + 2, rows0, g0, didx0, d0)

                wait_and_add(c1, rows1, g1, didx1, d1)

                @pl.when(c1 + 2 < CHW)
                def _():
                    fire(c1 + 2, rows1, g1, didx1, d1)

                return carry

            lax.fori_loop(0, CHW // 2, body, 0)

        @pl.when(cid == 0)
        def _():
            pipeline(CH0, sid * CH0)

        @pl.when(cid == 1)
        def _():
            pipeline(CH1, NS * CH0 + sid * CH1)

        plsc.subcore_barrier()
        pltpu.sync_copy(acc.at[pl.ds(sid * RPT, RPT)],
                        part.at[cid, pl.ds(sid * RPT, RPT)])

    return prop_kernel


def _make_prop1_kernel(CH, ACC_N, D):
    RPT = ACC_N // NS
    E_T = CH * K
    NB = 4

    @functools.partial(
        pl.kernel, mesh=_mesh,
        out_type=[jax.ShapeDtypeStruct((NC, ACC_N), jnp.float32)],
        scratch_types=[
            pltpu.VMEM((E_T,), jnp.int32),
            pltpu.VMEM((CH, K), jnp.int32),
            pltpu.VMEM((K,), jnp.float32),
            pltpu.VMEM((K,), jnp.float32),
            pltpu.VMEM((K,), jnp.float32),
            pltpu.VMEM((K,), jnp.float32),
            pltpu.VMEM((RPT,), jnp.float32),
            pltpu.VMEM_SHARED((ACC_N,), jnp.float32),
            pltpu.SemaphoreType.DMA,
            pltpu.SemaphoreType.DMA,
            pltpu.SemaphoreType.DMA,
            pltpu.SemaphoreType.DMA,
        ])
    def prop1_kernel(tflat_hbm, srcf_hbm, dst3_hbm, part,
                     sslab, dslab, v0, v1, v2, v3, zs, acc, s0, s1, s2, s3):
        cid = lax.axis_index("c")
        sid = lax.axis_index("s")
        wid = cid * NS + sid
        vals = [v0, v1, v2, v3]
        sems = [s0, s1, s2, s3]
        pltpu.sync_copy(srcf_hbm.at[pl.ds(wid * E_T, E_T)], sslab)
        pltpu.sync_copy(dst3_hbm.at[wid], dslab)

        def scale(c, carry):
            for j in range(K // 16):
                sl = pl.ds(c * K + 16 * j, 16)
                sslab[sl] = sslab[sl] * D
            return carry

        lax.fori_loop(0, CH, scale, 0)
        for j in range(RPT // 16):
            zs[pl.ds(16 * j, 16)] = jnp.zeros((16,), jnp.float32)
        pltpu.sync_copy(zs, acc.at[pl.ds(sid * RPT, RPT)])
        plsc.subcore_barrier()

        def fire(c, b):
            pltpu.async_copy(tflat_hbm.at[sslab.at[pl.ds(c * K, K)]],
                             vals[b], sems[b])

        def wait_and_add(c, b):
            pltpu.make_async_copy(tflat_hbm.at[sslab.at[pl.ds(c * K, K)]],
                                  vals[b], sems[b]).wait()
            pltpu.sync_copy(vals[b], acc.at[dslab.at[c]], add=True)

        for c in range(NB - 1):
            fire(c, c)

        def body(i, carry):
            for b in range(NB):
                c = NB * i + b
                wait_and_add(c, b)

                @pl.when(c + NB - 1 < CH)
                def _():
                    fire(c + NB - 1, (b + NB - 1) % NB)

            return carry

        lax.fori_loop(0, CH // NB, body, 0)
        plsc.subcore_barrier()
        pltpu.sync_copy(acc.at[pl.ds(sid * RPT, RPT)],
                        part.at[cid, pl.ds(sid * RPT, RPT)])

    return prop1_kernel


def _norm_from(deg):
    return jnp.where(deg > 0, lax.rsqrt(jnp.maximum(deg, 1.0)), 0.0)


def _mm1_body(x_ref, w_ref, dop_ref, out_ref):
    do = (dop_ref[0] + dop_ref[1]).reshape(-1)
    ns = _norm_from(do)
    h = jnp.dot(x_ref[...], w_ref[...], preferred_element_type=jnp.float32)
    out_ref[...] = h * ns[:, None]


def _mm2_body(aggp_ref, dip_ref, dop_ref, b1_ref, w2b_ref, out_ref):
    agg = aggp_ref[0] + aggp_ref[1]
    nd = _norm_from((dip_ref[0] + dip_ref[1]).reshape(-1))
    ns = _norm_from((dop_ref[0] + dop_ref[1]).reshape(-1))
    z = jnp.maximum(agg * nd[:, None] + b1_ref[0], 0.0)
    t = jnp.dot(z, w2b_ref[...], preferred_element_type=jnp.float32)
    out_ref[...] = t * ns[:, None]


def _fin_body(qp_ref, dip_ref, b2_ref, out_ref):
    q = qp_ref[0] + qp_ref[1]
    nd = _norm_from(dip_ref[0] + dip_ref[1])
    out_ref[...] = q * nd + b2_ref[0]


def kernel(x, edge_index, W1, b1, W2, b2):
    N, D = x.shape
    E = edge_index.shape[1]
    ACC_N = ((N + 1 + 2047) // 2048) * 2048
    CH = (E + NW * K - 1) // (NW * K)
    CH += (-CH) % 8
    EP = NW * K * CH
    NBR = 1024
    NB = ACC_N // NBR

    KP = 64
    CHT = (E + NS * KP - 1) // (NS * KP)
    CHT += CHT % 2
    CH1 = max(2, (CHT * 8) // 100 // 2 * 2)
    CH0 = CHT - CH1
    EPP = NS * KP * (CH0 + CH1)

    src = edge_index[0]
    dst = edge_index[1]
    pad = EP - E
    sent = jnp.full((pad,), N, jnp.int32)
    srcp = jnp.concatenate([src, sent])
    dstp = jnp.concatenate([dst, sent])
    src3 = srcp.reshape(NW, CH, K)
    dst3 = dstp.reshape(NW, CH, K)
    sent2 = jnp.full((EPP - E,), N, jnp.int32)
    srcf = jnp.concatenate([src, sent2])
    dstf = jnp.concatenate([dst, sent2])
    xp = jnp.pad(x, ((0, ACC_N - N), (0, 0)))

    dop, dip = _make_deg_kernel(CH, ACC_N)(src3, dst3)
    dop3 = dop.reshape(NC, ACC_N // 128, 128)
    dip3 = dip.reshape(NC, ACC_N // 128, 128)

    hs = pl.pallas_call(
        _mm1_body,
        grid=(NB,),
        in_specs=[
            pl.BlockSpec((NBR, D), lambda i: (i, 0)),
            pl.BlockSpec((D, D), lambda i: (0, 0)),
            pl.BlockSpec((NC, NBR // 128, 128), lambda i: (0, i, 0)),
        ],
        out_specs=pl.BlockSpec((NBR, D), lambda i: (i, 0)),
        out_shape=jax.ShapeDtypeStruct((ACC_N, D), jnp.float32),
    )(xp, W1, dop3)

    (part,) = _make_prop128_kernel(CH0, CH1, ACC_N, D, KP)(hs, srcf, dstf)

    w2b = jnp.tile(W2, (1, D))
    t2d = pl.pallas_call(
        _mm2_body,
        grid=(NB,),
        in_specs=[
            pl.BlockSpec((NC, NBR, D), lambda i: (0, i, 0)),
            pl.BlockSpec((NC, NBR // 128, 128), lambda i: (0, i, 0)),
            pl.BlockSpec((NC, NBR // 128, 128), lambda i: (0, i, 0)),
            pl.BlockSpec((1, D), lambda i: (0, 0)),
            pl.BlockSpec((D, D), lambda i: (0, 0)),
        ],
        out_specs=pl.BlockSpec((NBR, D), lambda i: (i, 0)),
        out_shape=jax.ShapeDtypeStruct((ACC_N, D), jnp.float32),
    )(part, dip3, dop3, b1.reshape(1, D), w2b)

    (part1,) = _make_prop1_kernel(CH, ACC_N, D)(t2d.reshape(-1), srcp, dst3)

    b2b = jnp.full((1, 128), b2[0], jnp.float32)
    outp = pl.pallas_call(
        _fin_body,
        grid=(ACC_N // 128 // 8,),
        in_specs=[
            pl.BlockSpec((NC, 8, 128), lambda i: (0, i, 0)),
            pl.BlockSpec((NC, 8, 128), lambda i: (0, i, 0)),
            pl.BlockSpec((1, 128), lambda i: (0, 0)),
        ],
        out_specs=pl.BlockSpec((8, 128), lambda i: (i, 0)),
        out_shape=jax.ShapeDtypeStruct((ACC_N // 128, 128), jnp.float32),
    )(part1.reshape(NC, ACC_N // 128, 128), dip3, b2b)

    return outp.reshape(-1)[:N].reshape(N, 1)
